# baseline (speedup 1.0000x reference)
import functools

import jax
import jax.numpy as jnp
from jax import lax
from jax.experimental import pallas as pl
from jax.experimental.pallas import tpu as pltpu
from jax.experimental.pallas import tpu_sc as plsc

B = 16384
N_FIELDS = 26
VOCAB = 100000
D = 64
GENRE = 10
NCOLS = N_FIELDS + GENRE
NOUT = N_FIELDS + 1
CB = 16
ROWS = CB * NOUT
L = 16


def _sc_call():
  info = plsc.get_sparse_core_info()
  nc, ns = info.num_cores, info.num_subcores
  nw = nc * ns
  per_w = B // nw
  n_chunks = per_w // CB
  mesh = plsc.VectorSubcoreMesh(core_axis_name="c", subcore_axis_name="s")

  @functools.partial(
      pl.kernel,
      mesh=mesh,
      compiler_params=pltpu.CompilerParams(use_tc_tiling_on_sc=False, needs_layout_passes=False),
      out_type=jax.ShapeDtypeStruct((B * NOUT, D), jnp.float32),
      scratch_types=[
          pltpu.VMEM((CB, NCOLS), jnp.int32),
          pltpu.VMEM((ROWS,), jnp.int32),
          pltpu.VMEM((ROWS, D), jnp.float32),
          pltpu.VMEM((GENRE, D), jnp.float32),
          pltpu.VMEM((CB, D), jnp.float32),
          pltpu.SemaphoreType.DMA,
      ],
  )
  def k(x_hbm, tab_hbm, ge_hbm, out_hbm,
        x_v, idx_v, rows_v, ge_v, gsc_v, sem):
    wid = lax.axis_index("s") * nc + lax.axis_index("c")
    base = wid * per_w
    pltpu.sync_copy(ge_hbm, ge_v)
    lane = lax.iota(jnp.int32, L)

    def chunk(i, carry):
      row0 = base + i * CB
      pltpu.sync_copy(x_hbm.at[pl.ds(row0, CB)], x_v)
      for t in range(ROWS // L):
        pos0 = t * L
        j0 = pos0 // NOUT
        cut = NOUT * (j0 + 1) - pos0
        if cut <= L - 1:
          j = jnp.where(lane >= cut, j0 + 1, j0)
        else:
          j = jnp.full((L,), j0, jnp.int32)
        f = pos0 + lane - j * NOUT
        xv = plsc.load_gather(x_v, [j, jnp.minimum(f, NCOLS - 1)])
        idx = jnp.where(f < N_FIELDS, f * VOCAB + xv, 0)
        idx_v[pl.ds(t * L, L)] = idx
      copies = []
      off = 0
      while off < ROWS:
        n = min(128, ROWS - off)
        copies.append(pltpu.async_copy(
            tab_hbm.at[idx_v.at[pl.ds(off, n)]],
            rows_v.at[pl.ds(off, n)], sem))
        off += n
      fg = jnp.minimum(lane + N_FIELDS, NCOLS - 1)
      for j in range(CB):
        jv = jnp.full((L,), j, jnp.int32)
        g = plsc.load_gather(x_v, [jv, fg]).astype(jnp.float32)
        g = jnp.where(lane < GENRE, g, 0.0)
        s = g
        for st in (1, 2, 4, 8):
          s = s + s.at[lane ^ st].get(mode="promise_in_bounds")
        w = g / s
        acc = [None] * (D // L)
        for kk in range(GENRE):
          wk = w.at[jnp.full((L,), kk, jnp.int32)].get(
              mode="promise_in_bounds")
          for c in range(D // L):
            term = wk * ge_v[kk, pl.ds(c * L, L)]
            acc[c] = term if acc[c] is None else acc[c] + term
        for c in range(D // L):
          gsc_v[j, pl.ds(c * L, L)] = acc[c]
      for cp in copies:
        cp.wait()
      for j in range(CB):
        for c in range(D // L):
          rows_v[j * NOUT + N_FIELDS, pl.ds(c * L, L)] = gsc_v[j, pl.ds(c * L, L)]
      pltpu.sync_copy(rows_v, out_hbm.at[pl.ds(row0 * NOUT, ROWS)])
      return carry

    lax.fori_loop(0, n_chunks, chunk, 0)

  return k


def kernel(x, tables, genre_embed):
  tab_flat = tables.reshape(N_FIELDS * VOCAB, D)
  out = _sc_call()(x, tab_flat, genre_embed)
  return out.reshape(B, NOUT, D)
